# baseline (speedup 1.0000x reference)
import functools

import jax
import jax.numpy as jnp
from jax import lax
from jax.experimental import pallas as pl
from jax.experimental.pallas import tpu as pltpu
from jax.experimental.pallas import tpu_sc as plsc

D_MODEL = 768
D_VOCAB = 100000
NC = 2
NS = 16
NW = NC * NS
T = 8192
BLKW = 256
NBLK = 390
TAIL0 = NBLK * BLKW
TAILB0 = D_VOCAB - BLKW
QD = 192
NQ = D_MODEL // QD
XCH = T // 16
RING = 4
SENTINEL = 0x7FFFFFFF


def _embed_body(w_hbm, tail_hbm, x_hbm, out_hbm, xtile, wlist, blist,
                qb0, qb1, stag, qs0, qs1, osem):
    wid = lax.axis_index("s") * NC + lax.axis_index("c")
    pltpu.sync_copy(x_hbm, xtile)

    iota = lax.iota(jnp.int32, 16)
    c0 = (wid * NBLK) >> 5
    c1 = ((wid + 1) * NBLK) >> 5

    def compact(dst, off, e, m):
        cs = plsc.cumsum(m.astype(jnp.int32))
        pos = jnp.where(m, off + cs - 1, 0)
        plsc.store_scatter(dst, [pos], e, mask=m)
        return off + cs[15]

    def scan_body(k, off):
        v = xtile[pl.ds(k * 16, 16)]
        e = (v << 13) | (iota + k * 16)
        blk = v >> 8
        return compact(wlist, off, e, (blk >= c0) & (blk < c1))

    wcount = lax.fori_loop(0, XCH, scan_body, 0)
    plsc.store_scatter(wlist, [wcount + iota],
                       jnp.full((16,), jnp.int32(SENTINEL)))
    plsc.store_scatter(wlist, [wcount + 16 + iota],
                       jnp.full((16,), jnp.int32(SENTINEL)))
    nwch = (wcount + 31) >> 4

    bufs = (qb0, qb1)
    sems = (qs0, qs1)

    def fire_q(cblk, q, buf, sem):
        src = w_hbm.at[pl.ds((q % NQ) * QD, QD),
                       pl.ds(pl.multiple_of(cblk * BLKW, 128), BLKW)]
        return pltpu.async_copy(src, buf, sem)

    def drain_q(buf, sem):
        pltpu.make_async_copy(
            w_hbm.at[pl.ds(0, QD), pl.ds(0, BLKW)], buf, sem).wait()

    def drain_out(slot, nwords):
        pltpu.make_async_copy(
            stag.at[pl.ds(slot * D_MODEL, nwords)],
            out_hbm.at[pl.ds(0, nwords)],
            osem.at[slot]).wait()

    fire_q(c0, 0, qb0, qs0)
    fire_q(c0, 1, qb1, qs1)

    def extract_rows(buf, nm, rows_per_tok, colv_fn, out_off_fn):
        def tok(i, drain):
            ch = blist[pl.ds((i >> 4) * 16, 16)]
            e = jnp.take_along_axis(
                ch, jnp.full((16,), i & 15, jnp.int32), axis=0)
            colv = colv_fn(e)
            t = e[0] & 8191
            slot = i & (RING - 1)
            sbase = slot * D_MODEL
            if drain:
                drain_out(slot, rows_per_tok)
            for j in range(rows_per_tok // 16):
                val = plsc.load_gather(buf, [iota + 16 * j, colv])
                stag[pl.ds(sbase + 16 * j, 16)] = val
            pltpu.async_copy(
                stag.at[pl.ds(pl.multiple_of(sbase, 8), rows_per_tok)],
                out_hbm.at[pl.ds(
                    pl.multiple_of(out_off_fn(t), 8), rows_per_tok)],
                osem.at[slot])
            return 0

        lax.fori_loop(0, jnp.minimum(nm, RING),
                      lambda i, c: tok(i, False), 0)
        lax.fori_loop(RING, jnp.maximum(nm, RING),
                      lambda i, c: tok(i, True), 0)
        lax.fori_loop(0, jnp.minimum(nm, RING),
                      lambda s, c: (drain_out(s, rows_per_tok), c)[1], 0)

    def block_body(cb, carry):
        def bl_body(k, off):
            ch = wlist[pl.ds(k * 16, 16)]
            return compact(blist, off, ch, (ch >> 21) == cb)

        nm = lax.fori_loop(0, nwch, bl_body, 0)

        for q in range(NQ):
            b = q & 1
            drain_q(bufs[b], sems[b])
            extract_rows(
                bufs[b], nm, QD,
                lambda e: (e >> 13) & (BLKW - 1),
                lambda t, q=q: t * D_MODEL + q * QD)
            nxt = q + 2
            fire_q(jnp.minimum(cb + nxt // NQ, NBLK - 1), nxt % NQ,
                   bufs[b], sems[b])
        return carry

    lax.fori_loop(c0, c1, block_body, 0)
    drain_q(qb0, qs0)
    drain_q(qb1, qs1)

    @pl.when(wid == NW - 1)
    def _tail():
        def tscan(k, off):
            v = xtile[pl.ds(k * 16, 16)]
            e = (v << 13) | (iota + k * 16)
            return compact(blist, off, e, v >= TAIL0)

        nm = lax.fori_loop(0, XCH, tscan, 0)

        for q in range(NQ):
            pltpu.sync_copy(tail_hbm.at[pl.ds(q * QD, QD)], qb0)
            extract_rows(
                qb0, nm, QD,
                lambda e: (e >> 13) - TAILB0,
                lambda t, q=q: t * D_MODEL + q * QD)


@functools.partial(
    pl.kernel,
    out_type=jax.ShapeDtypeStruct((T * D_MODEL,), jnp.float32),
    mesh=plsc.VectorSubcoreMesh(core_axis_name="c", subcore_axis_name="s"),
    compiler_params=pltpu.CompilerParams(needs_layout_passes=False),
    scratch_types=[
        pltpu.VMEM((T,), jnp.int32),
        pltpu.VMEM((T + 32,), jnp.int32),
        pltpu.VMEM((T + 32,), jnp.int32),
        pltpu.VMEM((QD, BLKW), jnp.float32),
        pltpu.VMEM((QD, BLKW), jnp.float32),
        pltpu.VMEM((RING * D_MODEL,), jnp.float32),
        pltpu.SemaphoreType.DMA,
        pltpu.SemaphoreType.DMA,
        pltpu.SemaphoreType.DMA((RING,)),
    ],
)
def _embed_call(w_hbm, tail_hbm, x_hbm, out_hbm, xtile, wlist, blist,
                qb0, qb1, stag, qs0, qs1, osem):
    _embed_body(w_hbm, tail_hbm, x_hbm, out_hbm, xtile, wlist, blist,
                qb0, qb1, stag, qs0, qs1, osem)


def kernel(x, W_E):
    b, s = x.shape
    xf = x.reshape(-1).astype(jnp.int32)
    tail = lax.slice(W_E, (0, TAILB0), (D_MODEL, D_VOCAB))
    out = _embed_call(W_E, tail, xf)
    return out.reshape(b, s, D_MODEL)
